# baseline (speedup 1.0000x reference)
import dataclasses
import functools

import jax
import jax.numpy as jnp
from jax import lax
from jax.experimental import pallas as pl
from jax.experimental.pallas import tpu as pltpu
from jax.experimental.pallas import tpu_sc as plsc

_DIM = 128
_NC = 2
_NS = 16
_NW = _NC * _NS
_L = 16
_TC_CH = 512


def _sc_gather_fn(b):
    bw = b // _NW
    mesh = plsc.VectorSubcoreMesh(core_axis_name="c", subcore_axis_name="s")
    cp = pltpu.CompilerParams()
    if "needs_layout_passes" in pltpu.CompilerParams.__dataclass_fields__:
        cp = dataclasses.replace(cp, needs_layout_passes=False)

    @functools.partial(
        pl.kernel,
        out_type=[
            jax.ShapeDtypeStruct((4 * b, _DIM), jnp.float32),
            jax.ShapeDtypeStruct((2 * b, _DIM), jnp.float32),
        ],
        mesh=mesh,
        scratch_types=[
            pltpu.VMEM((bw, 3), jnp.int32),
            pltpu.VMEM((bw, 3), jnp.int32),
            pltpu.VMEM((6 * bw,), jnp.int32),
            pltpu.VMEM((4 * bw, _DIM), jnp.float32),
            pltpu.SemaphoreType.DMA,
            pltpu.SemaphoreType.DMA,
        ],
        compiler_params=cp,
    )
    def gather(ent_hbm, rel_hbm, pos_hbm, neg_hbm, oe_hbm, or_hbm,
               pos_v, neg_v, idx_v, rows_v, gsem, osem):
        wid = lax.axis_index("s") * _NC + lax.axis_index("c")
        base = wid * bw
        pltpu.sync_copy(pos_hbm.at[pl.ds(base, bw)], pos_v)
        pltpu.sync_copy(neg_hbm.at[pl.ds(base, bw)], neg_v)

        lane = lax.iota(jnp.int32, _L)
        for slot, (src, col) in enumerate(
                [(pos_v, 0), (pos_v, 2), (neg_v, 0), (neg_v, 2),
                 (pos_v, 1), (neg_v, 1)]):
            colv = jnp.full((_L,), col, jnp.int32)
            for j in range(bw // _L):
                rowv = lane + (j * _L)
                idx_v[pl.ds(slot * bw + j * _L, _L)] = plsc.load_gather(
                    src, [rowv, colv])

        nbuf = 4

        def _gather(slot):
            table = ent_hbm if slot < 4 else rel_hbm
            buf = rows_v.at[pl.ds((slot % nbuf) * bw, bw)]
            return pltpu.async_copy(
                table.at[idx_v.at[pl.ds(slot * bw, bw)]], buf, gsem)

        def _write(slot):
            buf = rows_v.at[pl.ds((slot % nbuf) * bw, bw)]
            if slot < 4:
                dst = oe_hbm.at[pl.ds(slot * b + base, bw)]
            else:
                dst = or_hbm.at[pl.ds((slot - 4) * b + base, bw)]
            return pltpu.async_copy(buf, dst, osem)

        gathers = [_gather(s) for s in range(nbuf)]
        writes = {}
        gathers[0].wait()
        writes[0] = _write(0)
        gathers[1].wait()
        writes[1] = _write(1)
        writes[0].wait()
        gathers.append(_gather(4))
        gathers[2].wait()
        writes[2] = _write(2)
        writes[1].wait()
        gathers.append(_gather(5))
        gathers[3].wait()
        writes[3] = _write(3)
        gathers[4].wait()
        writes[4] = _write(4)
        gathers[5].wait()
        writes[5] = _write(5)
        for s in range(2, 6):
            writes[s].wait()

    return gather


def _unit(x):
    s = jnp.sum(x * x, axis=1, keepdims=True)
    return x * lax.rsqrt(jnp.maximum(s, 1e-24))


def _tc_loss_fn(inv_b):
    def _tc_loss(erows_ref, rrows_ref, out_ref):
        i = pl.program_id(0)
        hp = _unit(erows_ref[0])
        tp = _unit(erows_ref[1])
        hn = _unit(erows_ref[2])
        tn = _unit(erows_ref[3])
        rp = _unit(rrows_ref[0])
        rn = _unit(rrows_ref[1])
        dp = hp + rp - tp
        dn = hn + rn - tn
        ep = jnp.sqrt(jnp.sum(dp * dp, axis=1))
        en = jnp.sqrt(jnp.sum(dn * dn, axis=1))
        part = jnp.sum(jnp.maximum(1.0 + ep - en, 0.0))

        @pl.when(i == 0)
        def _():
            out_ref[...] = jnp.zeros((1, 1), jnp.float32)

        out_ref[...] += part.reshape(1, 1)

        @pl.when(i == pl.num_programs(0) - 1)
        def _():
            out_ref[...] *= inv_b

    return _tc_loss


@jax.jit
def kernel(pos_triples, neg_triples, ent_emb, rel_emb):
    b = pos_triples.shape[0]
    erows, rrows = _sc_gather_fn(b)(
        ent_emb, rel_emb, pos_triples, neg_triples)

    erows3 = erows.reshape(4, b, _DIM)
    rrows3 = rrows.reshape(2, b, _DIM)
    out = pl.pallas_call(
        _tc_loss_fn(1.0 / b),
        grid=(b // _TC_CH,),
        in_specs=[
            pl.BlockSpec((4, _TC_CH, _DIM), lambda i: (0, i, 0)),
            pl.BlockSpec((2, _TC_CH, _DIM), lambda i: (0, i, 0)),
        ],
        out_specs=pl.BlockSpec((1, 1), lambda i: (0, 0)),
        out_shape=jax.ShapeDtypeStruct((1, 1), jnp.float32),
    )(erows3, rrows3)
    return out[0, 0]


# speedup vs baseline: 1.0914x; 1.0914x over previous
import functools

import jax
import jax.numpy as jnp
from jax import lax
from jax.experimental import pallas as pl
from jax.experimental.pallas import tpu as pltpu
from jax.experimental.pallas import tpu_sc as plsc

_DIM = 128
_NC = 2
_NS = 16
_NW = _NC * _NS
_CHUNK = 128
_TC_CH = 512


def _sc_gather_fn(n_ent, n_rel):
    e_rows_w = n_ent // _NW
    r_rows_w = n_rel // _NW
    rows_w = e_rows_w + r_rows_w
    e_chunks = e_rows_w // _CHUNK
    r_chunks = r_rows_w // _CHUNK
    n_slots = e_chunks + r_chunks
    mesh = plsc.VectorSubcoreMesh(core_axis_name="c", subcore_axis_name="s")

    @functools.partial(
        pl.kernel,
        out_type=[
            jax.ShapeDtypeStruct((n_ent, _DIM), jnp.float32),
            jax.ShapeDtypeStruct((n_rel, _DIM), jnp.float32),
        ],
        mesh=mesh,
        scratch_types=[
            pltpu.VMEM((e_chunks, _CHUNK), jnp.int32),
            pltpu.VMEM((r_chunks, _CHUNK), jnp.int32),
            pltpu.VMEM((rows_w, _DIM), jnp.float32),
            pltpu.SemaphoreType.DMA,
            pltpu.SemaphoreType.DMA,
        ],
    )
    def gather(ent_hbm, rel_hbm, ie_hbm, ir_hbm, oe_hbm, or_hbm,
               ie_v, ir_v, rows_v, gsem, osem):
        wid = lax.axis_index("s") * _NC + lax.axis_index("c")
        pltpu.sync_copy(ie_hbm.at[pl.ds(wid * e_chunks, e_chunks)], ie_v)
        pltpu.sync_copy(ir_hbm.at[pl.ds(wid * r_chunks, r_chunks)], ir_v)
        gathers = []
        for j in range(e_chunks):
            gathers.append(pltpu.async_copy(
                ent_hbm.at[ie_v.at[j]],
                rows_v.at[pl.ds(j * _CHUNK, _CHUNK)], gsem))
        for j in range(r_chunks):
            gathers.append(pltpu.async_copy(
                rel_hbm.at[ir_v.at[j]],
                rows_v.at[pl.ds((e_chunks + j) * _CHUNK, _CHUNK)], gsem))
        writes = []
        for slot in range(n_slots):
            gathers[slot].wait()
            src = rows_v.at[pl.ds(slot * _CHUNK, _CHUNK)]
            if slot < e_chunks:
                dst = oe_hbm.at[pl.ds((wid * e_chunks + slot) * _CHUNK,
                                      _CHUNK)]
            else:
                k = slot - e_chunks
                dst = or_hbm.at[pl.ds((wid * r_chunks + k) * _CHUNK,
                                      _CHUNK)]
            writes.append(pltpu.async_copy(src, dst, osem))
        for wcopy in writes:
            wcopy.wait()

    return gather


def _unit(x):
    s = jnp.sum(x * x, axis=1, keepdims=True)
    return x * lax.rsqrt(jnp.maximum(s, 1e-24))


def _tc_loss_fn(inv_b):
    def _tc_loss(erows_ref, rrows_ref, out_ref):
        i = pl.program_id(0)
        hp = _unit(erows_ref[0])
        tp = _unit(erows_ref[1])
        hn = _unit(erows_ref[2])
        tn = _unit(erows_ref[3])
        rp = _unit(rrows_ref[0])
        rn = _unit(rrows_ref[1])
        dp = hp + rp - tp
        dn = hn + rn - tn
        ep = jnp.sqrt(jnp.sum(dp * dp, axis=1))
        en = jnp.sqrt(jnp.sum(dn * dn, axis=1))
        part = jnp.sum(jnp.maximum(1.0 + ep - en, 0.0))

        @pl.when(i == 0)
        def _():
            out_ref[...] = jnp.zeros((1, 1), jnp.float32)

        out_ref[...] += part.reshape(1, 1)

        @pl.when(i == pl.num_programs(0) - 1)
        def _():
            out_ref[...] *= inv_b

    return _tc_loss


@jax.jit
def kernel(pos_triples, neg_triples, ent_emb, rel_emb):
    b = pos_triples.shape[0]
    idx_ent = jnp.concatenate([
        pos_triples[:, 0], pos_triples[:, 2],
        neg_triples[:, 0], neg_triples[:, 2],
    ]).reshape(-1, _CHUNK)
    idx_rel = jnp.concatenate([
        pos_triples[:, 1], neg_triples[:, 1],
    ]).reshape(-1, _CHUNK)

    erows, rrows = _sc_gather_fn(4 * b, 2 * b)(
        ent_emb, rel_emb, idx_ent, idx_rel)

    erows3 = erows.reshape(4, b, _DIM)
    rrows3 = rrows.reshape(2, b, _DIM)
    out = pl.pallas_call(
        _tc_loss_fn(1.0 / b),
        grid=(b // _TC_CH,),
        in_specs=[
            pl.BlockSpec((4, _TC_CH, _DIM), lambda i: (0, i, 0)),
            pl.BlockSpec((2, _TC_CH, _DIM), lambda i: (0, i, 0)),
        ],
        out_specs=pl.BlockSpec((1, 1), lambda i: (0, 0)),
        out_shape=jax.ShapeDtypeStruct((1, 1), jnp.float32),
    )(erows3, rrows3)
    return out[0, 0]


# speedup vs baseline: 1.1552x; 1.0585x over previous
import functools

import jax
import jax.numpy as jnp
from jax import lax
from jax.experimental import pallas as pl
from jax.experimental.pallas import tpu as pltpu
from jax.experimental.pallas import tpu_sc as plsc

_DIM = 128
_NC = 2
_NS = 16
_NW = _NC * _NS
_CHUNK = 128
_TC_CH = 1024


def _sc_gather_fn(n_ent, n_rel):
    e_rows_w = n_ent // _NW
    r_rows_w = n_rel // _NW
    rows_w = e_rows_w + r_rows_w
    e_chunks = e_rows_w // _CHUNK
    r_chunks = r_rows_w // _CHUNK
    n_slots = e_chunks + r_chunks
    mesh = plsc.VectorSubcoreMesh(core_axis_name="c", subcore_axis_name="s")

    @functools.partial(
        pl.kernel,
        out_type=[
            jax.ShapeDtypeStruct((n_ent, _DIM), jnp.float32),
            jax.ShapeDtypeStruct((n_rel, _DIM), jnp.float32),
        ],
        mesh=mesh,
        scratch_types=[
            pltpu.VMEM((e_chunks, _CHUNK), jnp.int32),
            pltpu.VMEM((r_chunks, _CHUNK), jnp.int32),
            pltpu.VMEM((rows_w, _DIM), jnp.float32),
            pltpu.SemaphoreType.DMA,
            pltpu.SemaphoreType.DMA,
        ],
    )
    def gather(ent_hbm, rel_hbm, ie_hbm, ir_hbm, oe_hbm, or_hbm,
               ie_v, ir_v, rows_v, gsem, osem):
        wid = lax.axis_index("s") * _NC + lax.axis_index("c")
        pltpu.sync_copy(ie_hbm.at[pl.ds(wid * e_chunks, e_chunks)], ie_v)
        pltpu.sync_copy(ir_hbm.at[pl.ds(wid * r_chunks, r_chunks)], ir_v)
        gathers = []
        for j in range(e_chunks):
            gathers.append(pltpu.async_copy(
                ent_hbm.at[ie_v.at[j]],
                rows_v.at[pl.ds(j * _CHUNK, _CHUNK)], gsem))
        for j in range(r_chunks):
            gathers.append(pltpu.async_copy(
                rel_hbm.at[ir_v.at[j]],
                rows_v.at[pl.ds((e_chunks + j) * _CHUNK, _CHUNK)], gsem))
        writes = []
        for slot in range(n_slots):
            gathers[slot].wait()
            src = rows_v.at[pl.ds(slot * _CHUNK, _CHUNK)]
            if slot < e_chunks:
                dst = oe_hbm.at[pl.ds((wid * e_chunks + slot) * _CHUNK,
                                      _CHUNK)]
            else:
                k = slot - e_chunks
                dst = or_hbm.at[pl.ds((wid * r_chunks + k) * _CHUNK,
                                      _CHUNK)]
            writes.append(pltpu.async_copy(src, dst, osem))
        for wcopy in writes:
            wcopy.wait()

    return gather


def _unit(x):
    s = jnp.sum(x * x, axis=1)
    inv = lax.rsqrt(jnp.maximum(s, 1e-24))
    return x * inv[:, None]


def _tc_loss_fn(inv_b):
    def _tc_loss(erows_ref, rrows_ref, out_ref):
        i = pl.program_id(0)
        dp = _unit(erows_ref[0]) + _unit(rrows_ref[0]) - _unit(erows_ref[1])
        dn = _unit(erows_ref[2]) + _unit(rrows_ref[1]) - _unit(erows_ref[3])
        sp = jnp.maximum(jnp.sum(dp * dp, axis=1), 1e-30)
        sn = jnp.maximum(jnp.sum(dn * dn, axis=1), 1e-30)
        ep = sp * lax.rsqrt(sp)
        en = sn * lax.rsqrt(sn)
        part = jnp.sum(jnp.maximum(1.0 + ep - en, 0.0))

        @pl.when(i == 0)
        def _():
            out_ref[...] = jnp.zeros((1, 1), jnp.float32)

        out_ref[...] += part.reshape(1, 1)

        @pl.when(i == pl.num_programs(0) - 1)
        def _():
            out_ref[...] *= inv_b

    return _tc_loss


@jax.jit
def kernel(pos_triples, neg_triples, ent_emb, rel_emb):
    b = pos_triples.shape[0]
    idx_ent = jnp.concatenate([
        pos_triples[:, 0], pos_triples[:, 2],
        neg_triples[:, 0], neg_triples[:, 2],
    ]).reshape(-1, _CHUNK)
    idx_rel = jnp.concatenate([
        pos_triples[:, 1], neg_triples[:, 1],
    ]).reshape(-1, _CHUNK)

    erows, rrows = _sc_gather_fn(4 * b, 2 * b)(
        ent_emb, rel_emb, idx_ent, idx_rel)

    erows3 = erows.reshape(4, b, _DIM)
    rrows3 = rrows.reshape(2, b, _DIM)
    out = pl.pallas_call(
        _tc_loss_fn(1.0 / b),
        grid=(b // _TC_CH,),
        in_specs=[
            pl.BlockSpec((4, _TC_CH, _DIM), lambda i: (0, i, 0)),
            pl.BlockSpec((2, _TC_CH, _DIM), lambda i: (0, i, 0)),
        ],
        out_specs=pl.BlockSpec((1, 1), lambda i: (0, 0)),
        out_shape=jax.ShapeDtypeStruct((1, 1), jnp.float32),
    )(erows3, rrows3)
    return out[0, 0]


# speedup vs baseline: 1.1923x; 1.0321x over previous
import functools

import jax
import jax.numpy as jnp
from jax import lax
from jax.experimental import pallas as pl
from jax.experimental.pallas import tpu as pltpu
from jax.experimental.pallas import tpu_sc as plsc

_DIM = 128
_NC = 2
_NS = 16
_NW = _NC * _NS
_CHUNK = 128
_TC_CH = 2048


def _sc_gather_fn(n_ent, n_rel):
    e_rows_w = n_ent // _NW
    r_rows_w = n_rel // _NW
    rows_w = e_rows_w + r_rows_w
    e_chunks = e_rows_w // _CHUNK
    r_chunks = r_rows_w // _CHUNK
    n_slots = e_chunks + r_chunks
    mesh = plsc.VectorSubcoreMesh(core_axis_name="c", subcore_axis_name="s")

    @functools.partial(
        pl.kernel,
        out_type=[
            jax.ShapeDtypeStruct((n_ent, _DIM), jnp.float32),
            jax.ShapeDtypeStruct((n_rel, _DIM), jnp.float32),
        ],
        mesh=mesh,
        scratch_types=[
            pltpu.VMEM((e_chunks, _CHUNK), jnp.int32),
            pltpu.VMEM((r_chunks, _CHUNK), jnp.int32),
            pltpu.VMEM((rows_w, _DIM), jnp.float32),
            pltpu.SemaphoreType.DMA,
            pltpu.SemaphoreType.DMA,
        ],
    )
    def gather(ent_hbm, rel_hbm, ie_hbm, ir_hbm, oe_hbm, or_hbm,
               ie_v, ir_v, rows_v, gsem, osem):
        wid = lax.axis_index("s") * _NC + lax.axis_index("c")
        pltpu.sync_copy(ie_hbm.at[pl.ds(wid * e_chunks, e_chunks)], ie_v)
        pltpu.sync_copy(ir_hbm.at[pl.ds(wid * r_chunks, r_chunks)], ir_v)
        gathers = []
        for j in range(e_chunks):
            gathers.append(pltpu.async_copy(
                ent_hbm.at[ie_v.at[j]],
                rows_v.at[pl.ds(j * _CHUNK, _CHUNK)], gsem))
        for j in range(r_chunks):
            gathers.append(pltpu.async_copy(
                rel_hbm.at[ir_v.at[j]],
                rows_v.at[pl.ds((e_chunks + j) * _CHUNK, _CHUNK)], gsem))
        for g in gathers:
            g.wait()
        w0 = pltpu.async_copy(
            rows_v.at[pl.ds(0, e_rows_w)],
            oe_hbm.at[pl.ds(wid * e_rows_w, e_rows_w)], osem)
        w1 = pltpu.async_copy(
            rows_v.at[pl.ds(e_rows_w, r_rows_w)],
            or_hbm.at[pl.ds(wid * r_rows_w, r_rows_w)], osem)
        w0.wait()
        w1.wait()

    return gather


def _unit(x):
    s = jnp.sum(x * x, axis=1)
    inv = lax.rsqrt(jnp.maximum(s, 1e-24))
    return x * inv[:, None]


def _tc_loss_fn(inv_b):
    def _tc_loss(erows_ref, rrows_ref, out_ref):
        i = pl.program_id(0)
        dp = _unit(erows_ref[0]) + _unit(rrows_ref[0]) - _unit(erows_ref[1])
        dn = _unit(erows_ref[2]) + _unit(rrows_ref[1]) - _unit(erows_ref[3])
        sp = jnp.maximum(jnp.sum(dp * dp, axis=1), 1e-30)
        sn = jnp.maximum(jnp.sum(dn * dn, axis=1), 1e-30)
        ep = sp * lax.rsqrt(sp)
        en = sn * lax.rsqrt(sn)
        part = jnp.sum(jnp.maximum(1.0 + ep - en, 0.0))

        @pl.when(i == 0)
        def _():
            out_ref[...] = jnp.zeros((1, 1), jnp.float32)

        out_ref[...] += part.reshape(1, 1)

        @pl.when(i == pl.num_programs(0) - 1)
        def _():
            out_ref[...] *= inv_b

    return _tc_loss


@jax.jit
def kernel(pos_triples, neg_triples, ent_emb, rel_emb):
    b = pos_triples.shape[0]
    idx_ent = jnp.concatenate([
        pos_triples[:, 0], pos_triples[:, 2],
        neg_triples[:, 0], neg_triples[:, 2],
    ]).reshape(-1, _CHUNK)
    idx_rel = jnp.concatenate([
        pos_triples[:, 1], neg_triples[:, 1],
    ]).reshape(-1, _CHUNK)

    erows, rrows = _sc_gather_fn(4 * b, 2 * b)(
        ent_emb, rel_emb, idx_ent, idx_rel)

    erows3 = erows.reshape(4, b, _DIM)
    rrows3 = rrows.reshape(2, b, _DIM)
    out = pl.pallas_call(
        _tc_loss_fn(1.0 / b),
        grid=(b // _TC_CH,),
        in_specs=[
            pl.BlockSpec((4, _TC_CH, _DIM), lambda i: (0, i, 0)),
            pl.BlockSpec((2, _TC_CH, _DIM), lambda i: (0, i, 0)),
        ],
        out_specs=pl.BlockSpec((1, 1), lambda i: (0, 0)),
        out_shape=jax.ShapeDtypeStruct((1, 1), jnp.float32),
    )(erows3, rrows3)
    return out[0, 0]
